# baseline (speedup 1.0000x reference)
import functools

import jax
import jax.numpy as jnp
from jax import lax
from jax.experimental import pallas as pl
from jax.experimental.pallas import tpu as pltpu
from jax.experimental.pallas import tpu_sc as plsc

N = 50000
E = 800000
D_SCALAR = 64
D_EDGE = 16
D_U = 16
NDIM = 3
NUM_GRAPHS = 8
HID = 64

NC = 2
NS = 16
NW = NC * NS
CH = 128
NBUF = 5
K_CH = 200
PW = K_CH * CH
EP = NW * PW
TK = EP // CH
K0 = 360
K1 = TK // NS - K0
DP = 128
NP = 51200
RPT = NP // NS

S = 2
EC = E // S
TKC = TK // S
EPC = TKC * CH
K0C = 200
K1C = TKC // NS - K0C
BE = 3200
BN = 2000


def _sc_mesh():
    return plsc.VectorSubcoreMesh(
        core_axis_name="c", subcore_axis_name="s", num_cores=NC, num_subcores=NS
    )


def _sc_gather(table, idx2):
    @functools.partial(
        pl.kernel,
        out_type=jax.ShapeDtypeStruct((2 * EPC, DP), jnp.float32),
        mesh=_sc_mesh(),
        scratch_types=[
            pltpu.VMEM((NBUF, CH), jnp.int32),
            pltpu.VMEM((NBUF, CH, DP), jnp.float32),
            pltpu.SemaphoreType.DMA((NBUF,)),
            pltpu.SemaphoreType.DMA((NBUF,)),
        ],
        compiler_params=pltpu.CompilerParams(use_tc_tiling_on_sc=False),
    )
    def k(table_hbm, idx_hbm, out_hbm, idx_v, rows_v, gsem, wsem):
        c = lax.axis_index("c")
        s = lax.axis_index("s")
        cbase = jnp.where(c == 0, s * K0C, NS * K0C + s * K1C)
        ngrp = jnp.where(c == 0, K0C // NBUF, K1C // NBUF)

        def group(g, _):
            ep = g // ngrp
            ge = g - ep * ngrp
            pltpu.sync_copy(idx_hbm.at[ep, pl.ds(cbase + ge * NBUF, NBUF)], idx_v)
            gd = []
            for b in range(NBUF):
                @pl.when(g > 0)
                def _(b=b):
                    pltpu.make_async_copy(
                        rows_v.at[b], out_hbm.at[pl.ds(0, CH)], wsem.at[b]
                    ).wait()
                gd.append(pltpu.async_copy(
                    table_hbm.at[idx_v.at[b]], rows_v.at[b], gsem.at[b]
                ))
            for b in range(NBUF):
                gd[b].wait()
                pltpu.async_copy(
                    rows_v.at[b],
                    out_hbm.at[
                        pl.ds(ep * EPC + (cbase + ge * NBUF + b) * CH, CH)
                    ],
                    wsem.at[b],
                )
            return 0

        lax.fori_loop(0, 2 * ngrp, group, 0)
        for b in range(NBUF):
            @pl.when(ngrp > 0)
            def _(b=b):
                pltpu.make_async_copy(
                    rows_v.at[b], out_hbm.at[pl.ds(0, CH)], wsem.at[b]
                ).wait()

    return k(table, idx2)


def _sc_scatter(vals, idx_col3, zeros_np4):
    @functools.partial(
        pl.kernel,
        out_type=jax.ShapeDtypeStruct((NC, NP, 4), jnp.float32),
        mesh=_sc_mesh(),
        scratch_types=[
            pltpu.VMEM((8, CH), jnp.int32),
            pltpu.VMEM((8 * CH, 4), jnp.float32),
            pltpu.VMEM_SHARED((NP, 4), jnp.float32),
            pltpu.SemaphoreType.DMA((8,)),
        ],
        compiler_params=pltpu.CompilerParams(use_tc_tiling_on_sc=False),
    )
    def k(vals_hbm, idx_hbm, zeros_hbm, out_hbm, idx_v, v_v, acc, sems):
        c = lax.axis_index("c")
        s = lax.axis_index("s")
        wid = s * NC + c

        pltpu.sync_copy(zeros_hbm.at[pl.ds(s * RPT, RPT)], acc.at[pl.ds(s * RPT, RPT)])
        plsc.subcore_barrier()

        def group(g, _):
            pltpu.sync_copy(idx_hbm.at[pl.ds(wid * K_CH + g * 8, 8)], idx_v)
            pltpu.sync_copy(
                vals_hbm.at[pl.ds(wid * PW + g * 8 * CH, 8 * CH)], v_v
            )
            ds = [
                pltpu.async_copy(
                    v_v.at[pl.ds(b * CH, CH)],
                    acc.at[idx_v.at[b]],
                    sems.at[b],
                    add=True,
                )
                for b in range(8)
            ]
            for d in ds:
                d.wait()
            return 0

        lax.fori_loop(0, K_CH // 8, group, 0)
        plsc.subcore_barrier()
        pltpu.sync_copy(
            acc.at[pl.ds(s * RPT, RPT)], out_hbm.at[c, pl.ds(s * RPT, RPT)]
        )

    return k(vals, idx_col3, zeros_np4)


def _edge_body(gr_ref, gc_ref, ea_ref,
               sel_ref,
               w1r_ref, w1c_ref, w1ea_ref, w1rad_ref, b1e_ref, w2e_ref, b2e_ref,
               wah_ref, ba_ref,
               n1s_ref, n1f_ref, b1n_ref, w2n4_ref, b2n4_ref,
               *rest, aliased):
    ef_ref, ov_ref = rest[-2], rest[-1]
    gr = gr_ref[...]
    gc = gc_ref[...]
    d = gr - gc
    dd = d * d
    r3 = jnp.dot(dd, sel_ref[...], preferred_element_type=jnp.float32)
    lane3 = lax.broadcasted_iota(jnp.int32, (1, 3), 1)
    radial = jnp.where(lane3 < 2, jnp.sqrt(jnp.maximum(r3, 0.0)), r3)

    h1 = jnp.maximum(
        jnp.dot(gr, w1r_ref[...], preferred_element_type=jnp.float32)
        + jnp.dot(gc, w1c_ref[...], preferred_element_type=jnp.float32)
        + jnp.dot(ea_ref[...], w1ea_ref[...], preferred_element_type=jnp.float32)
        + jnp.dot(radial, w1rad_ref[...], preferred_element_type=jnp.float32)
        + b1e_ref[...],
        0.0,
    )
    e2 = jnp.dot(h1, w2e_ref[...], preferred_element_type=jnp.float32) + b2e_ref[...]
    att = jax.nn.sigmoid(
        jnp.dot(h1, wah_ref[...], preferred_element_type=jnp.float32) + ba_ref[...]
    )
    ef = e2 * att

    @pl.when(pl.program_id(0) < EC // BE)
    def _():
        ef_ref[...] = ef

    h2 = jnp.maximum(
        jnp.dot(gr, n1s_ref[...], preferred_element_type=jnp.float32)
        + jnp.dot(ef, n1f_ref[...], preferred_element_type=jnp.float32)
        + b1n_ref[...],
        0.0,
    )
    os4 = jnp.dot(h2, w2n4_ref[...], preferred_element_type=jnp.float32) + b2n4_ref[...]
    lane4 = lax.broadcasted_iota(jnp.int32, (1, 4), 1)
    ov = os4 * d[:, 80:84] + (lane4 == 3).astype(jnp.float32)
    i = pl.program_id(0)
    ov_ref[...] = jnp.where(i < EC // BE, ov, 0.0)


def _tc_edge(g_both, edge_attr, ew, s, ef_in, ov_in):
    grid = (EPC // BE,)
    nreal = EC // BE
    ncol = EPC // BE
    off_r = s * nreal
    off_p = s * ncol
    full = lambda shape: pl.BlockSpec(shape, lambda i: (0, 0))
    in_specs = [
        pl.BlockSpec((BE, DP), lambda i: (i, 0)),
        pl.BlockSpec((BE, DP), lambda i: (ncol + i, 0)),
        pl.BlockSpec((BE, D_EDGE),
                     lambda i: (off_r + jnp.minimum(i, nreal - 1), 0)),
        full((DP, 3)),
        full((DP, HID)), full((DP, HID)), full((D_EDGE, HID)), full((3, HID)),
        full((1, HID)), full((HID, HID)), full((1, HID)),
        full((HID, HID)), full((1, 1)),
        full((DP, HID)), full((HID, HID)), full((1, HID)),
        full((HID, 4)), full((1, 4)),
    ]
    args = [g_both, g_both, edge_attr, *ew]
    aliases = {}
    if ef_in is not None:
        anyspec = pl.BlockSpec(memory_space=pl.ANY)
        in_specs += [anyspec, anyspec]
        args += [ef_in, ov_in]
        aliases = {18: 0, 19: 1}
    return pl.pallas_call(
        functools.partial(_edge_body, aliased=ef_in is not None),
        grid=grid,
        in_specs=in_specs,
        out_specs=[
            pl.BlockSpec((BE, HID),
                         lambda i: (off_r + jnp.minimum(i, nreal - 1), 0)),
            pl.BlockSpec((BE, 4), lambda i: (off_p + i, 0)),
        ],
        out_shape=[
            jax.ShapeDtypeStruct((E, HID), jnp.float32),
            jax.ShapeDtypeStruct((EP, 4), jnp.float32),
        ],
        input_output_aliases=aliases,
    )(*args)


def _node_body(na_ref, bf_ref, parts_ref, u_ref,
               w2s_ref, w2v_ref, w2u_ref, b2_ref, w22_ref, b22_ref,
               wg1_ref, bg1_ref, wg2_ref, bg2_ref,
               out_ref, acc_ref):
    i = pl.program_id(0)
    nb = pl.num_programs(0)
    na = na_ref[...]
    nv = na[:, 0:6]
    ns = na[:, 6:70]
    bf = bf_ref[...]
    gids = lax.broadcasted_iota(jnp.int32, (1, NUM_GRAPHS), 1).astype(jnp.float32)
    onehot = (bf == gids).astype(jnp.float32)
    ub = jnp.dot(onehot, u_ref[...], preferred_element_type=jnp.float32)

    h = jnp.maximum(
        jnp.dot(ns, w2s_ref[...], preferred_element_type=jnp.float32)
        + jnp.dot(nv, w2v_ref[...], preferred_element_type=jnp.float32)
        + jnp.dot(ub, w2u_ref[...], preferred_element_type=jnp.float32)
        + b2_ref[...],
        0.0,
    )
    out_global = jnp.dot(h, w22_ref[...], preferred_element_type=jnp.float32) + b22_ref[...]

    sc = parts_ref[0] + parts_ref[1]
    out_graph = sc[:, 0:3] / jnp.maximum(sc[:, 3:4], 1.0)
    acc3 = out_global + out_graph

    gx = jnp.concatenate([acc3, na, jnp.ones_like(bf)], axis=1)
    partial = lax.dot_general(
        onehot, gx, (((0,), (0,)), ((), ())), preferred_element_type=jnp.float32
    )

    @pl.when(i == 0)
    def _():
        acc_ref[...] = partial

    @pl.when(i > 0)
    def _():
        acc_ref[...] += partial

    @pl.when(i == nb - 1)
    def _():
        g_sum = acc_ref[...]
        g_mean = g_sum[:, 0:73] / jnp.maximum(g_sum[:, 73:74], 1.0)
        gin = jnp.concatenate([u_ref[...], g_mean], axis=1)
        hg = jnp.maximum(
            jnp.dot(gin, wg1_ref[...], preferred_element_type=jnp.float32)
            + bg1_ref[...],
            0.0,
        )
        out_ref[...] = (
            jnp.dot(hg, wg2_ref[...], preferred_element_type=jnp.float32)
            + bg2_ref[...]
        )


def _tc_node(node_attr, batch_f, parts, u, nw):
    grid = (N // BN,)
    full2 = lambda shape: pl.BlockSpec(shape, lambda i: (0, 0))
    return pl.pallas_call(
        _node_body,
        grid=grid,
        in_specs=[
            pl.BlockSpec((BN, 2 * NDIM + D_SCALAR), lambda i: (i, 0)),
            pl.BlockSpec((BN, 1), lambda i: (i, 0)),
            pl.BlockSpec((NC, BN, 4), lambda i: (0, i, 0)),
            full2((NUM_GRAPHS, D_U)),
            full2((D_SCALAR, HID)), full2((6, HID)), full2((D_U, HID)),
            full2((1, HID)), full2((HID, NDIM)), full2((1, NDIM)),
            full2((D_U + NDIM + 2 * NDIM + D_SCALAR, HID)), full2((1, HID)),
            full2((HID, D_U)), full2((1, D_U)),
        ],
        out_specs=pl.BlockSpec((NUM_GRAPHS, D_U), lambda i: (0, 0)),
        out_shape=jax.ShapeDtypeStruct((NUM_GRAPHS, D_U), jnp.float32),
        scratch_shapes=[pltpu.VMEM((NUM_GRAPHS, 74), jnp.float32)],
    )(node_attr, batch_f, parts, u, *nw)


def kernel(node_attr, edge_index, edge_attr, u, batch, params):
    pos = node_attr[:, 0:3]
    vel = node_attr[:, 3:6]
    table = jnp.concatenate(
        [node_attr[:, 6:70],
         pos, vel, jnp.zeros((N, 2), jnp.float32),
         pos + vel, pos - vel, jnp.zeros((N, 2), jnp.float32),
         pos, jnp.zeros((N, DP - 83), jnp.float32)],
        axis=1,
    )
    row = edge_index[0].astype(jnp.int32)
    col = edge_index[1].astype(jnp.int32)
    rowp = jnp.pad(row.reshape(S, EC), ((0, 0), (0, EPC - EC))).reshape(S, TKC, CH)
    colp = jnp.pad(col.reshape(S, EC), ((0, 0), (0, EPC - EC))).reshape(S, TKC, CH)
    col2 = colp.reshape(TK, CH)

    pe = params["edge"]
    pa = params["att"]
    p1 = params["node1"]
    w1 = pe["W1"]
    zpad = jnp.zeros((DP - 64, HID), jnp.float32)
    sel = (
        jnp.zeros((DP, 3), jnp.float32)
        .at[64:67, 0].set(1.0)
        .at[67:70, 1].set(1.0)
        .at[72:75, 2].set(0.25)
        .at[75:78, 2].set(-0.25)
    )
    ew = (
        sel,
        jnp.concatenate([w1[0:64], zpad], axis=0),
        jnp.concatenate([w1[64:128], zpad], axis=0),
        w1[128:144],
        w1[144:147],
        pe["b1"].reshape(1, HID), pe["W2"], pe["b2"].reshape(1, HID),
        jnp.tile(pe["W2"] @ pa["W"], (1, HID)),
        (pe["b2"].reshape(1, HID) @ pa["W"] + pa["b"]).reshape(1, 1),
        jnp.concatenate([p1["W1"][0:64], zpad], axis=0),
        p1["W1"][64:128], p1["b1"].reshape(1, HID),
        jnp.tile(p1["W2"], (1, 4)),
        jnp.tile(p1["b2"].reshape(1, 1), (1, 4)),
    )
    idx2 = jnp.stack([rowp, colp], axis=1)
    ef, ov = None, None
    for s in range(S):
        g_both = _sc_gather(table, idx2[s])
        ef, ov = _tc_edge(g_both, edge_attr, ew, s, ef, ov)

    parts = _sc_scatter(ov, col2, jnp.zeros((NP, 4), jnp.float32))

    p2 = params["node2"]
    pg = params["glob"]
    w21 = p2["W1"]
    nw = (
        w21[0:64], w21[64:70], w21[70:86],
        p2["b1"].reshape(1, HID), p2["W2"], p2["b2"].reshape(1, NDIM),
        pg["W1"], pg["b1"].reshape(1, HID), pg["W2"], pg["b2"].reshape(1, D_U),
    )
    batch_f = batch.astype(jnp.float32).reshape(N, 1)
    u_new = _tc_node(node_attr, batch_f, parts, u, nw)

    return (node_attr, ef, u_new)


# speedup vs baseline: 1.0365x; 1.0365x over previous
import functools

import jax
import jax.numpy as jnp
from jax import lax
from jax.experimental import pallas as pl
from jax.experimental.pallas import tpu as pltpu
from jax.experimental.pallas import tpu_sc as plsc

N = 50000
E = 800000
D_SCALAR = 64
D_EDGE = 16
D_U = 16
NDIM = 3
NUM_GRAPHS = 8
HID = 64

NC = 2
NS = 16
NW = NC * NS
CH = 128
NBUF = 5
K_CH = 200
PW = K_CH * CH
EP = NW * PW
TK = EP // CH
K0 = 360
K1 = TK // NS - K0
DP = 128
NP = 51200
RPT = NP // NS

S = 2
EC = E // S
TKC = TK // S
EPC = TKC * CH
K0C = 190
K1C = TKC // NS - K0C
BE = 3200
BN = 2000


def _sc_mesh():
    return plsc.VectorSubcoreMesh(
        core_axis_name="c", subcore_axis_name="s", num_cores=NC, num_subcores=NS
    )


def _sc_gather(table, idx2):
    @functools.partial(
        pl.kernel,
        out_type=jax.ShapeDtypeStruct((2 * EPC, DP), jnp.float32),
        mesh=_sc_mesh(),
        scratch_types=[
            pltpu.VMEM((NBUF, CH), jnp.int32),
            pltpu.VMEM((NBUF, CH, DP), jnp.float32),
            pltpu.SemaphoreType.DMA((NBUF,)),
            pltpu.SemaphoreType.DMA((NBUF,)),
        ],
        compiler_params=pltpu.CompilerParams(use_tc_tiling_on_sc=False),
    )
    def k(table_hbm, idx_hbm, out_hbm, idx_v, rows_v, gsem, wsem):
        c = lax.axis_index("c")
        s = lax.axis_index("s")
        cbase = jnp.where(c == 0, s * K0C, NS * K0C + s * K1C)
        ngrp = jnp.where(c == 0, K0C // NBUF, K1C // NBUF)

        def group(g, _):
            ep = g // ngrp
            ge = g - ep * ngrp
            pltpu.sync_copy(idx_hbm.at[ep, pl.ds(cbase + ge * NBUF, NBUF)], idx_v)
            gd = []
            for b in range(NBUF):
                @pl.when(g > 0)
                def _(b=b):
                    pltpu.make_async_copy(
                        rows_v.at[b], out_hbm.at[pl.ds(0, CH)], wsem.at[b]
                    ).wait()
                gd.append(pltpu.async_copy(
                    table_hbm.at[idx_v.at[b]], rows_v.at[b], gsem.at[b]
                ))
            for b in range(NBUF):
                gd[b].wait()
                pltpu.async_copy(
                    rows_v.at[b],
                    out_hbm.at[
                        pl.ds(ep * EPC + (cbase + ge * NBUF + b) * CH, CH)
                    ],
                    wsem.at[b],
                )
            return 0

        lax.fori_loop(0, 2 * ngrp, group, 0)
        for b in range(NBUF):
            @pl.when(ngrp > 0)
            def _(b=b):
                pltpu.make_async_copy(
                    rows_v.at[b], out_hbm.at[pl.ds(0, CH)], wsem.at[b]
                ).wait()

    return k(table, idx2)


def _sc_scatter(vals, idx_col3, zeros_np4):
    @functools.partial(
        pl.kernel,
        out_type=jax.ShapeDtypeStruct((NC, NP, 4), jnp.float32),
        mesh=_sc_mesh(),
        scratch_types=[
            pltpu.VMEM((8, CH), jnp.int32),
            pltpu.VMEM((8 * CH, 4), jnp.float32),
            pltpu.VMEM_SHARED((NP, 4), jnp.float32),
            pltpu.SemaphoreType.DMA((8,)),
        ],
        compiler_params=pltpu.CompilerParams(use_tc_tiling_on_sc=False),
    )
    def k(vals_hbm, idx_hbm, zeros_hbm, out_hbm, idx_v, v_v, acc, sems):
        c = lax.axis_index("c")
        s = lax.axis_index("s")
        wid = s * NC + c

        pltpu.sync_copy(zeros_hbm.at[pl.ds(s * RPT, RPT)], acc.at[pl.ds(s * RPT, RPT)])
        plsc.subcore_barrier()

        def group(g, _):
            pltpu.sync_copy(idx_hbm.at[pl.ds(wid * K_CH + g * 8, 8)], idx_v)
            pltpu.sync_copy(
                vals_hbm.at[pl.ds(wid * PW + g * 8 * CH, 8 * CH)], v_v
            )
            ds = [
                pltpu.async_copy(
                    v_v.at[pl.ds(b * CH, CH)],
                    acc.at[idx_v.at[b]],
                    sems.at[b],
                    add=True,
                )
                for b in range(8)
            ]
            for d in ds:
                d.wait()
            return 0

        lax.fori_loop(0, K_CH // 8, group, 0)
        plsc.subcore_barrier()
        pltpu.sync_copy(
            acc.at[pl.ds(s * RPT, RPT)], out_hbm.at[c, pl.ds(s * RPT, RPT)]
        )

    return k(vals, idx_col3, zeros_np4)


def _edge_body(gr_ref, gc_ref, ea_ref,
               sel_ref,
               w1r_ref, w1c_ref, w1ea_ref, w1rad_ref, b1e_ref, w2e_ref, b2e_ref,
               wah_ref, ba_ref,
               n1s_ref, n1f_ref, b1n_ref, w2n4_ref, b2n4_ref,
               *rest, aliased):
    ef_ref, ov_ref = rest[-2], rest[-1]
    gr = gr_ref[...]
    gc = gc_ref[...]
    d = gr - gc
    dd = d * d
    r3 = jnp.dot(dd, sel_ref[...], preferred_element_type=jnp.float32)
    lane3 = lax.broadcasted_iota(jnp.int32, (1, 3), 1)
    radial = jnp.where(lane3 < 2, jnp.sqrt(jnp.maximum(r3, 0.0)), r3)

    h1 = jnp.maximum(
        jnp.dot(gr, w1r_ref[...], preferred_element_type=jnp.float32)
        + jnp.dot(gc, w1c_ref[...], preferred_element_type=jnp.float32)
        + jnp.dot(ea_ref[...], w1ea_ref[...], preferred_element_type=jnp.float32)
        + jnp.dot(radial, w1rad_ref[...], preferred_element_type=jnp.float32)
        + b1e_ref[...],
        0.0,
    )
    e2 = jnp.dot(h1, w2e_ref[...], preferred_element_type=jnp.float32) + b2e_ref[...]
    att = jax.nn.sigmoid(
        jnp.dot(h1, wah_ref[...], preferred_element_type=jnp.float32) + ba_ref[...]
    )
    ef = e2 * att

    @pl.when(pl.program_id(0) < EC // BE)
    def _():
        ef_ref[...] = ef

    h2 = jnp.maximum(
        jnp.dot(gr, n1s_ref[...], preferred_element_type=jnp.float32)
        + jnp.dot(ef, n1f_ref[...], preferred_element_type=jnp.float32)
        + b1n_ref[...],
        0.0,
    )
    os4 = jnp.dot(h2, w2n4_ref[...], preferred_element_type=jnp.float32) + b2n4_ref[...]
    lane4 = lax.broadcasted_iota(jnp.int32, (1, 4), 1)
    ov = os4 * d[:, 80:84] + (lane4 == 3).astype(jnp.float32)
    i = pl.program_id(0)
    ov_ref[...] = jnp.where(i < EC // BE, ov, 0.0)


def _tc_edge(g_both, edge_attr, ew, s, ef_in, ov_in):
    grid = (EPC // BE,)
    nreal = EC // BE
    ncol = EPC // BE
    off_r = s * nreal
    off_p = s * ncol
    full = lambda shape: pl.BlockSpec(shape, lambda i: (0, 0))
    in_specs = [
        pl.BlockSpec((BE, DP), lambda i: (i, 0)),
        pl.BlockSpec((BE, DP), lambda i: (ncol + i, 0)),
        pl.BlockSpec((BE, D_EDGE),
                     lambda i: (off_r + jnp.minimum(i, nreal - 1), 0)),
        full((DP, 3)),
        full((DP, HID)), full((DP, HID)), full((D_EDGE, HID)), full((3, HID)),
        full((1, HID)), full((HID, HID)), full((1, HID)),
        full((HID, HID)), full((1, 1)),
        full((DP, HID)), full((HID, HID)), full((1, HID)),
        full((HID, 4)), full((1, 4)),
    ]
    args = [g_both, g_both, edge_attr, *ew]
    aliases = {}
    if ef_in is not None:
        anyspec = pl.BlockSpec(memory_space=pl.ANY)
        in_specs += [anyspec, anyspec]
        args += [ef_in, ov_in]
        aliases = {18: 0, 19: 1}
    return pl.pallas_call(
        functools.partial(_edge_body, aliased=ef_in is not None),
        grid=grid,
        in_specs=in_specs,
        out_specs=[
            pl.BlockSpec((BE, HID),
                         lambda i: (off_r + jnp.minimum(i, nreal - 1), 0)),
            pl.BlockSpec((BE, 4), lambda i: (off_p + i, 0)),
        ],
        out_shape=[
            jax.ShapeDtypeStruct((E, HID), jnp.float32),
            jax.ShapeDtypeStruct((EP, 4), jnp.float32),
        ],
        input_output_aliases=aliases,
    )(*args)


def _node_body(na_ref, bf_ref, parts_ref, u_ref,
               w2s_ref, w2v_ref, w2u_ref, b2_ref, w22_ref, b22_ref,
               wg1_ref, bg1_ref, wg2_ref, bg2_ref,
               out_ref, acc_ref):
    i = pl.program_id(0)
    nb = pl.num_programs(0)
    na = na_ref[...]
    nv = na[:, 0:6]
    ns = na[:, 6:70]
    bf = bf_ref[...]
    gids = lax.broadcasted_iota(jnp.int32, (1, NUM_GRAPHS), 1).astype(jnp.float32)
    onehot = (bf == gids).astype(jnp.float32)
    ub = jnp.dot(onehot, u_ref[...], preferred_element_type=jnp.float32)

    h = jnp.maximum(
        jnp.dot(ns, w2s_ref[...], preferred_element_type=jnp.float32)
        + jnp.dot(nv, w2v_ref[...], preferred_element_type=jnp.float32)
        + jnp.dot(ub, w2u_ref[...], preferred_element_type=jnp.float32)
        + b2_ref[...],
        0.0,
    )
    out_global = jnp.dot(h, w22_ref[...], preferred_element_type=jnp.float32) + b22_ref[...]

    sc = parts_ref[0] + parts_ref[1]
    out_graph = sc[:, 0:3] / jnp.maximum(sc[:, 3:4], 1.0)
    acc3 = out_global + out_graph

    gx = jnp.concatenate([acc3, na, jnp.ones_like(bf)], axis=1)
    partial = lax.dot_general(
        onehot, gx, (((0,), (0,)), ((), ())), preferred_element_type=jnp.float32
    )

    @pl.when(i == 0)
    def _():
        acc_ref[...] = partial

    @pl.when(i > 0)
    def _():
        acc_ref[...] += partial

    @pl.when(i == nb - 1)
    def _():
        g_sum = acc_ref[...]
        g_mean = g_sum[:, 0:73] / jnp.maximum(g_sum[:, 73:74], 1.0)
        gin = jnp.concatenate([u_ref[...], g_mean], axis=1)
        hg = jnp.maximum(
            jnp.dot(gin, wg1_ref[...], preferred_element_type=jnp.float32)
            + bg1_ref[...],
            0.0,
        )
        out_ref[...] = (
            jnp.dot(hg, wg2_ref[...], preferred_element_type=jnp.float32)
            + bg2_ref[...]
        )


def _tc_node(node_attr, batch_f, parts, u, nw):
    grid = (N // BN,)
    full2 = lambda shape: pl.BlockSpec(shape, lambda i: (0, 0))
    return pl.pallas_call(
        _node_body,
        grid=grid,
        in_specs=[
            pl.BlockSpec((BN, 2 * NDIM + D_SCALAR), lambda i: (i, 0)),
            pl.BlockSpec((BN, 1), lambda i: (i, 0)),
            pl.BlockSpec((NC, BN, 4), lambda i: (0, i, 0)),
            full2((NUM_GRAPHS, D_U)),
            full2((D_SCALAR, HID)), full2((6, HID)), full2((D_U, HID)),
            full2((1, HID)), full2((HID, NDIM)), full2((1, NDIM)),
            full2((D_U + NDIM + 2 * NDIM + D_SCALAR, HID)), full2((1, HID)),
            full2((HID, D_U)), full2((1, D_U)),
        ],
        out_specs=pl.BlockSpec((NUM_GRAPHS, D_U), lambda i: (0, 0)),
        out_shape=jax.ShapeDtypeStruct((NUM_GRAPHS, D_U), jnp.float32),
        scratch_shapes=[pltpu.VMEM((NUM_GRAPHS, 74), jnp.float32)],
    )(node_attr, batch_f, parts, u, *nw)


def kernel(node_attr, edge_index, edge_attr, u, batch, params):
    pos = node_attr[:, 0:3]
    vel = node_attr[:, 3:6]
    table = jnp.concatenate(
        [node_attr[:, 6:70],
         pos, vel, jnp.zeros((N, 2), jnp.float32),
         pos + vel, pos - vel, jnp.zeros((N, 2), jnp.float32),
         pos, jnp.zeros((N, DP - 83), jnp.float32)],
        axis=1,
    )
    row = edge_index[0].astype(jnp.int32)
    col = edge_index[1].astype(jnp.int32)
    rowp = jnp.pad(row.reshape(S, EC), ((0, 0), (0, EPC - EC))).reshape(S, TKC, CH)
    colp = jnp.pad(col.reshape(S, EC), ((0, 0), (0, EPC - EC))).reshape(S, TKC, CH)
    col2 = colp.reshape(TK, CH)

    pe = params["edge"]
    pa = params["att"]
    p1 = params["node1"]
    w1 = pe["W1"]
    zpad = jnp.zeros((DP - 64, HID), jnp.float32)
    sel = (
        jnp.zeros((DP, 3), jnp.float32)
        .at[64:67, 0].set(1.0)
        .at[67:70, 1].set(1.0)
        .at[72:75, 2].set(0.25)
        .at[75:78, 2].set(-0.25)
    )
    ew = (
        sel,
        jnp.concatenate([w1[0:64], zpad], axis=0),
        jnp.concatenate([w1[64:128], zpad], axis=0),
        w1[128:144],
        w1[144:147],
        pe["b1"].reshape(1, HID), pe["W2"], pe["b2"].reshape(1, HID),
        jnp.tile(pe["W2"] @ pa["W"], (1, HID)),
        (pe["b2"].reshape(1, HID) @ pa["W"] + pa["b"]).reshape(1, 1),
        jnp.concatenate([p1["W1"][0:64], zpad], axis=0),
        p1["W1"][64:128], p1["b1"].reshape(1, HID),
        jnp.tile(p1["W2"], (1, 4)),
        jnp.tile(p1["b2"].reshape(1, 1), (1, 4)),
    )
    idx2 = jnp.stack([rowp, colp], axis=1)
    ef, ov = None, None
    for s in range(S):
        g_both = _sc_gather(table, idx2[s])
        ef, ov = _tc_edge(g_both, edge_attr, ew, s, ef, ov)

    parts = _sc_scatter(ov, col2, jnp.zeros((NP, 4), jnp.float32))

    p2 = params["node2"]
    pg = params["glob"]
    w21 = p2["W1"]
    nw = (
        w21[0:64], w21[64:70], w21[70:86],
        p2["b1"].reshape(1, HID), p2["W2"], p2["b2"].reshape(1, NDIM),
        pg["W1"], pg["b1"].reshape(1, HID), pg["W2"], pg["b2"].reshape(1, D_U),
    )
    batch_f = batch.astype(jnp.float32).reshape(N, 1)
    u_new = _tc_node(node_attr, batch_f, parts, u, nw)

    return (node_attr, ef, u_new)
